# baseline (speedup 1.0000x reference)
import functools

import jax
import jax.numpy as jnp
from jax import lax
from jax.experimental import pallas as pl
from jax.experimental.pallas import tpu as pltpu
from jax.experimental.pallas import tpu_sc as plsc

_DIM = 1024
_SEQ = 8192
_SCALE = _DIM ** (-0.5)
_NC = 2
_NS = 16
_NW = _NC * _NS
_CHUNK = 32
_NBUF = 3
_LOOK = 2
_LANES = 16

_TC_ROWS = 4096

_mesh = plsc.VectorSubcoreMesh(core_axis_name="c", subcore_axis_name="s")


def _make_sc(row0, n_rows):
    rpw = n_rows // _NW
    nchunks = rpw // _CHUNK
    assert rpw % _CHUNK == 0

    @functools.partial(
        pl.kernel,
        out_type=jax.ShapeDtypeStruct((n_rows, _DIM), jnp.float32),
        mesh=_mesh,
        scratch_types=[
            pltpu.VMEM((_NBUF, _CHUNK, _DIM), jnp.float32),
            [pltpu.SemaphoreType.DMA] * _NBUF,
            [pltpu.SemaphoreType.DMA] * _NBUF,
        ],
    )
    def _sc_scale(emb_hbm, out_hbm, buf, sin, sout):
        wid = lax.axis_index("s") * _NC + lax.axis_index("c")
        base = wid * rpw

        def start_in(c):
            b = c % _NBUF
            return pltpu.async_copy(
                emb_hbm.at[pl.ds(row0 + base + c * _CHUNK, _CHUNK)],
                buf.at[b], sin[b])

        def start_out(c):
            b = c % _NBUF
            return pltpu.async_copy(
                buf.at[b], out_hbm.at[pl.ds(base + c * _CHUNK, _CHUNK)],
                sout[b])

        def scale_buf(b):
            def body(r, carry):
                for j in range(_DIM // _LANES):
                    idx = (b, r, pl.ds(j * _LANES, _LANES))
                    buf[idx] = buf[idx] * _SCALE
                return carry
            lax.fori_loop(0, _CHUNK, body, 0, unroll=False)

        d_in = {c: start_in(c) for c in range(min(_LOOK, nchunks))}
        d_out = {}
        out_pending = []
        for c in range(nchunks):
            d_in[c].wait()
            if c + _LOOK < nchunks:
                prev = c + _LOOK - _NBUF
                if prev >= 0:
                    d_out[prev].wait()
                    out_pending.remove(prev)
                d_in[c + _LOOK] = start_in(c + _LOOK)
            scale_buf(c % _NBUF)
            d_out[c] = start_out(c)
            out_pending.append(c)
        for c in out_pending:
            d_out[c].wait()

    return _sc_scale


_sc_tail = _make_sc(_TC_ROWS, _SEQ - _TC_ROWS)


def _tc_body(emb_ref, o_ref):
    o_ref[...] = emb_ref[...] * _SCALE


def _tc_head(emb):
    block_rows = 512
    return pl.pallas_call(
        _tc_body,
        grid=(_TC_ROWS // block_rows,),
        in_specs=[pl.BlockSpec((block_rows, _DIM), lambda i: (i, 0))],
        out_specs=pl.BlockSpec((block_rows, _DIM), lambda i: (i, 0)),
        out_shape=jax.ShapeDtypeStruct((_TC_ROWS, _DIM), jnp.float32),
    )(emb)


def kernel(x, emb):
    tail = _sc_tail(emb)
    head = _tc_head(emb)
    return jnp.concatenate([head, tail], axis=0)


# speedup vs baseline: 1.3442x; 1.3442x over previous
import functools

import jax
import jax.numpy as jnp
from jax import lax
from jax.experimental import pallas as pl
from jax.experimental.pallas import tpu as pltpu
from jax.experimental.pallas import tpu_sc as plsc

_DIM = 1024
_SEQ = 8192
_SCALE = _DIM ** (-0.5)
_NC = 2
_NS = 16
_NW = _NC * _NS
_RPW = _SEQ // _NW
_CHUNK = 32
_NCHUNKS = _RPW // _CHUNK
_NBUF = 3
_LOOK = 2
_LANES = 16

_mesh = plsc.VectorSubcoreMesh(core_axis_name="c", subcore_axis_name="s")


@functools.partial(
    pl.kernel,
    out_type=jax.ShapeDtypeStruct((_SEQ, _DIM), jnp.float32),
    mesh=_mesh,
    scratch_types=[
        pltpu.VMEM((_NBUF, _CHUNK, _DIM), jnp.float32),
        [pltpu.SemaphoreType.DMA] * _NBUF,
        [pltpu.SemaphoreType.DMA] * _NBUF,
    ],
)
def _sc_scale(emb_hbm, out_hbm, buf, sin, sout):
    wid = lax.axis_index("s") * _NC + lax.axis_index("c")
    base = wid * _RPW

    def start_in(c):
        b = c % _NBUF
        return pltpu.async_copy(
            emb_hbm.at[pl.ds(base + c * _CHUNK, _CHUNK)], buf.at[b], sin[b])

    def start_out(c):
        b = c % _NBUF
        return pltpu.async_copy(
            buf.at[b], out_hbm.at[pl.ds(base + c * _CHUNK, _CHUNK)], sout[b])

    def scale_buf(b):
        def body(r, carry):
            for j in range(_DIM // _LANES):
                idx = (b, r, pl.ds(j * _LANES, _LANES))
                buf[idx] = buf[idx] * _SCALE
            return carry
        lax.fori_loop(0, _CHUNK, body, 0, unroll=False)

    d_in = {c: start_in(c) for c in range(min(_LOOK, _NCHUNKS))}
    d_out = {}
    out_pending = []
    for c in range(_NCHUNKS):
        d_in[c].wait()
        scale_buf(c % _NBUF)
        if c + _LOOK < _NCHUNKS:
            prev = c + _LOOK - _NBUF
            if prev >= 0:
                d_out[prev].wait()
                out_pending.remove(prev)
            d_in[c + _LOOK] = start_in(c + _LOOK)
        d_out[c] = start_out(c)
        out_pending.append(c)
    for c in out_pending:
        d_out[c].wait()


def kernel(x, emb):
    seq_len = x.shape[1]
    return _sc_scale(emb[:seq_len])
